# baseline (speedup 1.0000x reference)
import functools

import jax
import jax.numpy as jnp
import numpy as np
from jax import lax
from jax.experimental import pallas as pl
from jax.experimental.pallas import tpu as pltpu
from jax.experimental.pallas import tpu_sc as plsc


_VID = np.zeros((4, 16), dtype=np.int32)
_VID[0, 0:3] = [0, 1, 2]
_VID[1, 0:12] = [0, 4, 3, 1, 0, 1, 1, 0, 2, 3, 5, 4]
_VID[2] = [0, 1, 2, 3, 0, 1, 2, 3, 4, 5, 6, 7, 4, 5, 6, 7]
_VID[3, 0:12] = [0, 0, 1, 1, 2, 2, 3, 3, 4, 4, 5, 5]


def _select_chain(vid, values):
    out = jnp.full((16,), values[-1], jnp.float32)
    for t in range(len(values) - 2, -1, -1):
        out = jnp.where(vid == t, jnp.full((16,), values[t], jnp.float32), out)
    return out


@functools.partial(
    pl.kernel,
    out_type=jax.ShapeDtypeStruct((64,), jnp.float32),
    mesh=plsc.VectorSubcoreMesh(core_axis_name="c", subcore_axis_name="s"),
    scratch_types=[
        pltpu.VMEM((16,), jnp.float32),
        pltpu.VMEM((2, 64), jnp.float32),
        pltpu.VMEM((2, 3, 128), jnp.float32),
        pltpu.VMEM((4, 16), jnp.int32),
        pltpu.VMEM((64,), jnp.float32),
    ],
)
def _gather_kernel(x_hbm, y_hbm, z_hbm, vid_hbm, out_hbm,
                   xbuf, ybuf, zbuf, vidbuf, obuf):
    cid = lax.axis_index("c")
    sid = lax.axis_index("s")

    @pl.when(jnp.logical_and(cid == 0, sid == 0))
    def _():
        pltpu.sync_copy(x_hbm.at[pl.ds(0, 16)], xbuf)
        pltpu.sync_copy(y_hbm.at[pl.ds(0, 2)], ybuf)
        pltpu.sync_copy(z_hbm.at[pl.ds(0, 2), pl.ds(0, 3)], zbuf)
        pltpu.sync_copy(vid_hbm, vidbuf)

        vx = xbuf[...]
        vy0 = ybuf[0, pl.ds(0, 16)]
        vy1 = ybuf[1, pl.ds(0, 16)]
        vz = [[zbuf[i, j, pl.ds(0, 16)] for j in range(3)] for i in range(2)]

        va = _select_chain(vidbuf[0], [vx[2], vx[0], vx[1]])
        vy = _select_chain(
            vidbuf[1], [vy0[0], vy0[1], vy0[2], vy1[0], vy1[1], vy1[2]])
        vd = _select_chain(
            vidbuf[2], [vz[i][0][k] for i in range(2) for k in range(4)])
        ve = _select_chain(
            vidbuf[3], [vz[i][j][0] for i in range(2) for j in range(3)])

        obuf[pl.ds(0, 16)] = va
        obuf[pl.ds(16, 16)] = vy
        obuf[pl.ds(32, 16)] = vd
        obuf[pl.ds(48, 16)] = ve
        pltpu.sync_copy(obuf, out_hbm)


def kernel(x, y, z):
    out = _gather_kernel(x, y, z, jnp.asarray(_VID))
    a = out[0:3]
    b = out[16:22].reshape(3, 2)
    c = out[22:28].reshape(2, 3)
    d = out[32:48].reshape(2, 2, 4)
    e = out[48:60].reshape(2, 3, 2)
    return (a, b, c, d, e)


# speedup vs baseline: 2.4241x; 2.4241x over previous
import functools

import jax
import jax.numpy as jnp
from jax import lax
from jax.experimental import pallas as pl
from jax.experimental.pallas import tpu as pltpu
from jax.experimental.pallas import tpu_sc as plsc

_F32 = jnp.float32


@functools.partial(
    pl.kernel,
    out_type=(
        jax.ShapeDtypeStruct((3,), _F32),
        jax.ShapeDtypeStruct((6,), _F32),
        jax.ShapeDtypeStruct((6,), _F32),
        jax.ShapeDtypeStruct((16,), _F32),
        jax.ShapeDtypeStruct((12,), _F32),
    ),
    mesh=plsc.VectorSubcoreMesh(
        core_axis_name="c", subcore_axis_name="s", num_cores=1),
    scratch_types=[
        pltpu.VMEM((16,), _F32),
        pltpu.VMEM((2, 16), _F32),
        pltpu.VMEM((2, 3, 16), _F32),
        pltpu.VMEM((16,), _F32),
        pltpu.VMEM((16,), _F32),
        pltpu.VMEM((16,), _F32),
        pltpu.VMEM((16,), _F32),
        pltpu.VMEM((16,), _F32),
        pltpu.SemaphoreType.DMA,
    ],
)
def _gather_kernel(x_hbm, y_hbm, z_hbm,
                   a_hbm, b_hbm, c_hbm, d_hbm, e_hbm,
                   xbuf, ybuf, zbuf, abuf, bbuf, cbuf, dbuf, ebuf, sem):
    sid = lax.axis_index("s")

    @pl.when(sid == 0)
    def _():
        pltpu.async_copy(x_hbm.at[pl.ds(0, 16)], xbuf, sem)
        pltpu.async_copy(y_hbm, ybuf, sem)
        pltpu.async_copy(z_hbm, zbuf, sem).wait()
        pltpu.make_async_copy(x_hbm.at[pl.ds(0, 16)], xbuf, sem).wait()
        pltpu.make_async_copy(y_hbm, ybuf, sem).wait()

        lane = lax.iota(jnp.int32, 16)
        vx = xbuf[...]
        vy0 = ybuf[0, :]
        vy1 = ybuf[1, :]
        vz00 = zbuf[0, 0, :]
        vz10 = zbuf[1, 0, :]

        def bcast(s):
            return jnp.full((16,), s, _F32)

        def lanechain(values):
            out = bcast(values[-1])
            for t in range(len(values) - 2, -1, -1):
                out = jnp.where(lane == t, bcast(values[t]), out)
            return out

        abuf[...] = lanechain([vx[2], vx[0], vx[1]])
        bbuf[...] = lanechain(
            [vy0[0], vy1[1], vy1[0], vy0[1], vy0[0], vy0[1]])
        cbuf[...] = lanechain(
            [vy0[1], vy0[0], vy0[2], vy1[0], vy1[2], vy1[1]])
        dbuf[...] = lanechain(
            [vz00[0], vz00[1], vz00[2], vz00[3]] * 2
            + [vz10[0], vz10[1], vz10[2], vz10[3]] * 2)
        ev = [bcast(zbuf[i, j, :][0]) for i in range(2) for j in range(3)]
        evec = ev[5]
        for t in range(4, -1, -1):
            evec = jnp.where(lane < 2 * (t + 1), ev[t], evec)
        ebuf[...] = evec

        pltpu.async_copy(abuf.at[pl.ds(0, 3)], a_hbm, sem)
        pltpu.async_copy(bbuf.at[pl.ds(0, 6)], b_hbm, sem)
        pltpu.async_copy(cbuf.at[pl.ds(0, 6)], c_hbm, sem)
        pltpu.async_copy(dbuf, d_hbm, sem)
        pltpu.async_copy(ebuf.at[pl.ds(0, 12)], e_hbm, sem).wait()
        pltpu.make_async_copy(abuf.at[pl.ds(0, 3)], a_hbm, sem).wait()
        pltpu.make_async_copy(bbuf.at[pl.ds(0, 6)], b_hbm, sem).wait()
        pltpu.make_async_copy(cbuf.at[pl.ds(0, 6)], c_hbm, sem).wait()
        pltpu.make_async_copy(dbuf, d_hbm, sem).wait()


def kernel(x, y, z):
    y2 = lax.slice(y, (0, 0), (2, 16))
    z2 = lax.slice(z, (0, 0, 0), (2, 3, 16))
    a, b, c, d, e = _gather_kernel(x, y2, z2)
    return (a, b.reshape(3, 2), c.reshape(2, 3),
            d.reshape(2, 2, 4), e.reshape(2, 3, 2))
